# baseline (speedup 1.0000x reference)
import functools

import jax
import jax.numpy as jnp
from jax import lax
from jax.experimental import pallas as pl
from jax.experimental.pallas import tpu as pltpu, tpu_sc as plsc


def _make_gather(B: int, D: int):
    info = plsc.get_sparse_core_info()
    NC, NS = info.num_cores, info.num_subcores
    NW = NC * NS
    assert B % (8 * NW) == 0
    b_per_w = B // NW
    mesh = plsc.VectorSubcoreMesh(core_axis_name="c", subcore_axis_name="s")

    @functools.partial(
        pl.kernel,
        out_type=(
            jax.ShapeDtypeStruct((B, D), jnp.float32),
            jax.ShapeDtypeStruct((B, D), jnp.float32),
            jax.ShapeDtypeStruct((B,), jnp.int32),
        ),
        mesh=mesh,
        scratch_types=[
            pltpu.VMEM((b_per_w,), jnp.int32),
            pltpu.VMEM((b_per_w, D), jnp.float32),
            pltpu.VMEM((b_per_w, D), jnp.float32),
            pltpu.VMEM((b_per_w,), jnp.int32),
            pltpu.SemaphoreType.DMA,
        ],
        compiler_params=pltpu.CompilerParams(use_tc_tiling_on_sc=False),
    )
    def gather(mem_hbm, pos_hbm, lu_hbm, idx_hbm,
               z_hbm, pos_z_hbm, lu_out_hbm,
               idx_v, z_v, pos_v, lu_v, sem):
        wid = lax.axis_index("s") * NC + lax.axis_index("c")
        base = wid * b_per_w
        pltpu.sync_copy(idx_hbm.at[pl.ds(base, b_per_w)], idx_v)
        c1 = pltpu.make_async_copy(mem_hbm.at[idx_v], z_v, sem)
        c2 = pltpu.make_async_copy(pos_hbm.at[idx_v], pos_v, sem)
        c3 = pltpu.make_async_copy(lu_hbm.at[idx_v], lu_v, sem)
        c1.start()
        c2.start()
        c3.start()
        c1.wait()
        c2.wait()
        c3.wait()
        pltpu.sync_copy(z_v, z_hbm.at[pl.ds(base, b_per_w)])
        pltpu.sync_copy(pos_v, pos_z_hbm.at[pl.ds(base, b_per_w)])
        pltpu.sync_copy(lu_v, lu_out_hbm.at[pl.ds(base, b_per_w)])

    return gather


def kernel(memory, pos_memory, last_update, n_id):
    B = n_id.shape[0]
    D = memory.shape[1]
    gather = _make_gather(B, D)
    n_id = n_id.astype(jnp.int32)
    z, pos_z, lu = gather(memory, pos_memory, last_update, n_id)
    return (z, pos_z, lu)


# speedup vs baseline: 1.4788x; 1.4788x over previous
import functools

import jax
import jax.numpy as jnp
from jax import lax
from jax.experimental import pallas as pl
from jax.experimental.pallas import tpu as pltpu, tpu_sc as plsc

_CHUNK = 16


def _make_gather(B: int, D: int, with_lu: bool):
    info = plsc.get_sparse_core_info()
    NC, NS = info.num_cores, info.num_subcores
    NW = NC * NS
    assert B % (8 * NW) == 0
    b_per_w = B // NW
    assert b_per_w % _CHUNK == 0
    mesh = plsc.VectorSubcoreMesh(core_axis_name="c", subcore_axis_name="s")

    out_type = [jax.ShapeDtypeStruct((B, D), jnp.float32)]
    scratch = [
        pltpu.VMEM((b_per_w,), jnp.int32),
        pltpu.VMEM((_CHUNK, 8, D), jnp.float32),
        pltpu.VMEM((b_per_w, D), jnp.float32),
        pltpu.SemaphoreType.DMA,
    ]
    if with_lu:
        out_type.append(jax.ShapeDtypeStruct((B,), jnp.int32))
        scratch += [
            pltpu.VMEM((b_per_w,), jnp.int32),
            pltpu.SemaphoreType.DMA,
        ]

    @functools.partial(
        pl.kernel,
        out_type=tuple(out_type),
        mesh=mesh,
        scratch_types=scratch,
    )
    def gather(table_hbm, *rest):
        if with_lu:
            (lu_hbm, idx_hbm, out_hbm, lu_out_hbm,
             idx_v, stage_v, rows_v, sem, lu_v, lsem) = rest
        else:
            (idx_hbm, out_hbm,
             idx_v, stage_v, rows_v, sem) = rest
        wid = lax.axis_index("s") * NC + lax.axis_index("c")
        base = wid * b_per_w
        pltpu.sync_copy(idx_hbm.at[pl.ds(base, b_per_w)], idx_v)

        if with_lu:
            lu_cp = pltpu.make_async_copy(lu_hbm.at[idx_v], lu_v, lsem)
            lu_cp.start()

        def chunk_body(ci, _):
            c = pl.multiple_of(ci * _CHUNK, _CHUNK)
            iv = idx_v[pl.ds(c, 16)]

            for l in range(16):
                i = iv[l]
                g = pl.multiple_of((i // 8) * 8, 8)
                pltpu.make_async_copy(
                    table_hbm.at[pl.ds(g, 8)], stage_v.at[l], sem).start()

            pltpu.make_async_copy(
                table_hbm.at[pl.ds(0, 8 * _CHUNK)].reshape(_CHUNK, 8, D),
                stage_v, sem).wait()

            for l in range(16):
                r = iv[l] % 8
                for k in range(0, D, 16):
                    rows_v[c + l, pl.ds(k, 16)] = stage_v[l, r, pl.ds(k, 16)]
            return 0

        lax.fori_loop(0, b_per_w // _CHUNK, chunk_body, 0)

        pltpu.sync_copy(rows_v, out_hbm.at[pl.ds(base, b_per_w)])
        if with_lu:
            lu_cp.wait()
            pltpu.sync_copy(lu_v, lu_out_hbm.at[pl.ds(base, b_per_w)])

    return gather


def kernel(memory, pos_memory, last_update, n_id):
    B = n_id.shape[0]
    D = memory.shape[1]
    n_id = n_id.astype(jnp.int32)
    z, lu = _make_gather(B, D, True)(memory, last_update, n_id)
    (pos_z,) = _make_gather(B, D, False)(pos_memory, n_id)
    return (z, pos_z, lu)


# speedup vs baseline: 1.4794x; 1.0004x over previous
import functools

import jax
import jax.numpy as jnp
from jax import lax
from jax.experimental import pallas as pl
from jax.experimental.pallas import tpu as pltpu, tpu_sc as plsc

_CHUNK = 16


def _make_gather(B: int, D: int, with_lu: bool):
    info = plsc.get_sparse_core_info()
    NC, NS = info.num_cores, info.num_subcores
    NW = NC * NS
    assert B % (8 * NW) == 0
    b_per_w = B // NW
    assert b_per_w % _CHUNK == 0
    mesh = plsc.VectorSubcoreMesh(core_axis_name="c", subcore_axis_name="s")

    out_type = [jax.ShapeDtypeStruct((B, D), jnp.float32)]
    scratch = [
        pltpu.VMEM((b_per_w,), jnp.int32),
        pltpu.VMEM((_CHUNK, 8, D), jnp.float32),
        pltpu.VMEM((b_per_w, D), jnp.float32),
        pltpu.SemaphoreType.DMA,
    ]
    if with_lu:
        out_type.append(jax.ShapeDtypeStruct((B,), jnp.int32))
        scratch += [
            pltpu.VMEM((b_per_w,), jnp.int32),
            pltpu.SemaphoreType.DMA,
        ]

    @functools.partial(
        pl.kernel,
        out_type=tuple(out_type),
        mesh=mesh,
        scratch_types=scratch,
        compiler_params=pltpu.CompilerParams(skip_device_barrier=True),
    )
    def gather(table_hbm, *rest):
        if with_lu:
            (lu_hbm, idx_hbm, out_hbm, lu_out_hbm,
             idx_v, stage_v, rows_v, sem, lu_v, lsem) = rest
        else:
            (idx_hbm, out_hbm,
             idx_v, stage_v, rows_v, sem) = rest
        wid = lax.axis_index("s") * NC + lax.axis_index("c")
        base = wid * b_per_w
        pltpu.sync_copy(idx_hbm.at[pl.ds(base, b_per_w)], idx_v)

        if with_lu:
            lu_cp = pltpu.make_async_copy(lu_hbm.at[idx_v], lu_v, lsem)
            lu_cp.start()

        def chunk_body(ci, _):
            c = pl.multiple_of(ci * _CHUNK, _CHUNK)
            iv = idx_v[pl.ds(c, 16)]

            for l in range(16):
                i = iv[l]
                g = pl.multiple_of((i // 8) * 8, 8)
                pltpu.make_async_copy(
                    table_hbm.at[pl.ds(g, 8)], stage_v.at[l], sem).start()

            pltpu.make_async_copy(
                table_hbm.at[pl.ds(0, 8 * _CHUNK)].reshape(_CHUNK, 8, D),
                stage_v, sem).wait()

            for l in range(16):
                r = iv[l] % 8
                for k in range(0, D, 16):
                    rows_v[c + l, pl.ds(k, 16)] = stage_v[l, r, pl.ds(k, 16)]
            return 0

        lax.fori_loop(0, b_per_w // _CHUNK, chunk_body, 0)

        pltpu.sync_copy(rows_v, out_hbm.at[pl.ds(base, b_per_w)])
        if with_lu:
            lu_cp.wait()
            pltpu.sync_copy(lu_v, lu_out_hbm.at[pl.ds(base, b_per_w)])

    return gather


def kernel(memory, pos_memory, last_update, n_id):
    B = n_id.shape[0]
    D = memory.shape[1]
    n_id = n_id.astype(jnp.int32)
    z, lu = _make_gather(B, D, True)(memory, last_update, n_id)
    (pos_z,) = _make_gather(B, D, False)(pos_memory, n_id)
    return (z, pos_z, lu)
